# initial kernel scaffold (unmeasured)
import jax
import jax.numpy as jnp
from jax import lax
from jax.experimental import pallas as pl
from jax.experimental.pallas import tpu as pltpu


def kernel(
    x,
):
    def body(*refs):
        pass

    out_shape = jax.ShapeDtypeStruct(..., jnp.float32)
    return pl.pallas_call(body, out_shape=out_shape)(...)


# baseline (device time: 31010 ns/iter reference)
import jax
import jax.numpy as jnp
from jax import lax
from jax.experimental import pallas as pl
from jax.experimental.pallas import tpu as pltpu

M, N = 512, 512


def kernel(x):
    x2d = x.reshape(M, N)

    def body(x_ref, out_ref, comm_ref, send_sems, recv_sems):
        my_x = lax.axis_index("x")
        my_y = lax.axis_index("y")
        y_nbr = (my_x, 1 - my_y)
        x_nbr = (1 - my_x, my_y)

        barrier_sem = pltpu.get_barrier_semaphore()
        for nbr in (y_nbr, x_nbr):
            pl.semaphore_signal(
                barrier_sem, inc=1,
                device_id=nbr, device_id_type=pl.DeviceIdType.MESH,
            )
        pl.semaphore_wait(barrier_sem, 2)

        rdma1 = pltpu.make_async_remote_copy(
            src_ref=x_ref,
            dst_ref=comm_ref.at[0],
            send_sem=send_sems.at[0],
            recv_sem=recv_sems.at[0],
            device_id=y_nbr,
            device_id_type=pl.DeviceIdType.MESH,
        )
        rdma1.start()
        rdma1.wait()
        out_ref[...] = x_ref[...] + comm_ref[0]

        rdma2 = pltpu.make_async_remote_copy(
            src_ref=out_ref,
            dst_ref=comm_ref.at[1],
            send_sem=send_sems.at[1],
            recv_sem=recv_sems.at[1],
            device_id=x_nbr,
            device_id_type=pl.DeviceIdType.MESH,
        )
        rdma2.start()
        rdma2.wait()
        out_ref[...] = out_ref[...] + comm_ref[1]

    return pl.pallas_call(
        body,
        out_shape=jax.ShapeDtypeStruct((M, N), jnp.float32),
        in_specs=[pl.BlockSpec(memory_space=pltpu.VMEM)],
        out_specs=pl.BlockSpec(memory_space=pltpu.VMEM),
        scratch_shapes=[
            pltpu.VMEM((2, M, N), jnp.float32),
            pltpu.SemaphoreType.DMA((2,)),
            pltpu.SemaphoreType.DMA((2,)),
        ],
        compiler_params=pltpu.CompilerParams(collective_id=0),
    )(x2d)


# device time: 20273 ns/iter; 1.5296x vs baseline; 1.5296x over previous
import jax
import jax.numpy as jnp
from jax import lax
from jax.experimental import pallas as pl
from jax.experimental.pallas import tpu as pltpu

M, N = 512, 512
H, Q, E = 256, 128, 64


def kernel(x):
    x2d = x.reshape(M, N)

    def body(x_ref, out_ref, rA1, rB1, rA2, rB2, S, R):
        my_x = lax.axis_index("x")
        my_y = lax.axis_index("y")
        y_nbr = (my_x, 1 - my_y)
        x_nbr = (1 - my_x, my_y)

        barrier_sem = pltpu.get_barrier_semaphore()
        for nbr in (y_nbr, x_nbr):
            pl.semaphore_signal(
                barrier_sem, inc=1,
                device_id=nbr, device_id_type=pl.DeviceIdType.MESH,
            )
        pl.semaphore_wait(barrier_sem, 2)

        aq_own = my_y * Q
        aq_oth = (1 - my_y) * Q
        ae_own = aq_own + my_x * E
        ae_oth = aq_own + (1 - my_x) * E
        bq_own = H + my_x * Q
        bq_oth = H + (1 - my_x) * Q
        be_own = bq_own + my_y * E
        be_oth = bq_own + (1 - my_y) * E

        def mrc(src, dst, i, nbr):
            return pltpu.make_async_remote_copy(
                src_ref=src, dst_ref=dst,
                send_sem=S.at[i], recv_sem=R.at[i],
                device_id=nbr, device_id_type=pl.DeviceIdType.MESH,
            )

        a1 = mrc(x_ref.at[pl.ds(aq_oth, Q), :], rA1, 0, y_nbr)
        b1 = mrc(x_ref.at[pl.ds(bq_oth, Q), :], rB1, 1, x_nbr)
        a1.start()
        b1.start()
        a1.wait()
        out_ref[pl.ds(aq_own, Q), :] = x_ref[pl.ds(aq_own, Q), :] + rA1[...]
        b1.wait()
        out_ref[pl.ds(bq_own, Q), :] = x_ref[pl.ds(bq_own, Q), :] + rB1[...]

        a2 = mrc(out_ref.at[pl.ds(ae_oth, E), :], rA2, 2, x_nbr)
        b2 = mrc(out_ref.at[pl.ds(be_oth, E), :], rB2, 3, y_nbr)
        a2.start()
        b2.start()
        a2.wait()
        out_ref[pl.ds(ae_own, E), :] = out_ref[pl.ds(ae_own, E), :] + rA2[...]
        b2.wait()
        out_ref[pl.ds(be_own, E), :] = out_ref[pl.ds(be_own, E), :] + rB2[...]

        a3 = mrc(out_ref.at[pl.ds(ae_own, E), :],
                 out_ref.at[pl.ds(ae_own, E), :], 4, x_nbr)
        b3 = mrc(out_ref.at[pl.ds(be_own, E), :],
                 out_ref.at[pl.ds(be_own, E), :], 5, y_nbr)
        a3.start()
        b3.start()
        a3.wait()
        b3.wait()

        a4 = mrc(out_ref.at[pl.ds(aq_own, Q), :],
                 out_ref.at[pl.ds(aq_own, Q), :], 6, y_nbr)
        b4 = mrc(out_ref.at[pl.ds(bq_own, Q), :],
                 out_ref.at[pl.ds(bq_own, Q), :], 7, x_nbr)
        a4.start()
        b4.start()
        a4.wait()
        b4.wait()

    return pl.pallas_call(
        body,
        out_shape=jax.ShapeDtypeStruct((M, N), jnp.float32),
        in_specs=[pl.BlockSpec(memory_space=pltpu.VMEM)],
        out_specs=pl.BlockSpec(memory_space=pltpu.VMEM),
        scratch_shapes=[
            pltpu.VMEM((Q, N), jnp.float32),
            pltpu.VMEM((Q, N), jnp.float32),
            pltpu.VMEM((E, N), jnp.float32),
            pltpu.VMEM((E, N), jnp.float32),
            pltpu.SemaphoreType.DMA((8,)),
            pltpu.SemaphoreType.DMA((8,)),
        ],
        compiler_params=pltpu.CompilerParams(collective_id=0),
    )(x2d)


# device time: 19055 ns/iter; 1.6274x vs baseline; 1.0639x over previous
import jax
import jax.numpy as jnp
from jax import lax
from jax.experimental import pallas as pl
from jax.experimental.pallas import tpu as pltpu

M, N = 512, 512
H, Q, E = 256, 128, 64


def kernel(x):
    x2d = x.reshape(M, N)

    def body(x_ref, out_ref, rA1, rB1, rA2, rB2, S, R):
        my_x = lax.axis_index("x")
        my_y = lax.axis_index("y")
        y_nbr = (my_x, 1 - my_y)
        x_nbr = (1 - my_x, my_y)

        barrier_sem = pltpu.get_barrier_semaphore()
        for nbr in (y_nbr, x_nbr):
            pl.semaphore_signal(
                barrier_sem, inc=1,
                device_id=nbr, device_id_type=pl.DeviceIdType.MESH,
            )
        pl.semaphore_wait(barrier_sem, 2)

        aq_own = my_y * Q
        aq_oth = (1 - my_y) * Q
        ae_own = aq_own + my_x * E
        ae_oth = aq_own + (1 - my_x) * E
        bq_own = H + my_x * Q
        bq_oth = H + (1 - my_x) * Q
        be_own = bq_own + my_y * E
        be_oth = bq_own + (1 - my_y) * E

        def mrc(src, dst, i, nbr):
            return pltpu.make_async_remote_copy(
                src_ref=src, dst_ref=dst,
                send_sem=S.at[i], recv_sem=R.at[i],
                device_id=nbr, device_id_type=pl.DeviceIdType.MESH,
            )

        a1f = mrc(x_ref.at[pl.ds(aq_oth + (1 - my_x) * E, E), :],
                  rA1.at[0], 0, y_nbr)
        a1s = mrc(x_ref.at[pl.ds(aq_oth + my_x * E, E), :],
                  rA1.at[1], 1, y_nbr)
        b1f = mrc(x_ref.at[pl.ds(bq_oth + (1 - my_y) * E, E), :],
                  rB1.at[0], 2, x_nbr)
        b1s = mrc(x_ref.at[pl.ds(bq_oth + my_y * E, E), :],
                  rB1.at[1], 3, x_nbr)
        a1f.start()
        b1f.start()
        a1s.start()
        b1s.start()

        a1f.wait()
        out_ref[pl.ds(ae_oth, E), :] = x_ref[pl.ds(ae_oth, E), :] + rA1[0]
        a2 = mrc(out_ref.at[pl.ds(ae_oth, E), :], rA2, 4, x_nbr)
        a2.start()

        b1f.wait()
        out_ref[pl.ds(be_oth, E), :] = x_ref[pl.ds(be_oth, E), :] + rB1[0]
        b2 = mrc(out_ref.at[pl.ds(be_oth, E), :], rB2, 5, y_nbr)
        b2.start()

        a1s.wait()
        out_ref[pl.ds(ae_own, E), :] = x_ref[pl.ds(ae_own, E), :] + rA1[1]
        b1s.wait()
        out_ref[pl.ds(be_own, E), :] = x_ref[pl.ds(be_own, E), :] + rB1[1]

        a2.wait()
        out_ref[pl.ds(ae_own, E), :] = out_ref[pl.ds(ae_own, E), :] + rA2[...]
        a3 = mrc(out_ref.at[pl.ds(ae_own, E), :],
                 out_ref.at[pl.ds(ae_own, E), :], 6, x_nbr)
        a4a = mrc(out_ref.at[pl.ds(ae_own, E), :],
                  out_ref.at[pl.ds(ae_own, E), :], 8, y_nbr)
        a3.start()
        a4a.start()

        b2.wait()
        out_ref[pl.ds(be_own, E), :] = out_ref[pl.ds(be_own, E), :] + rB2[...]
        b3 = mrc(out_ref.at[pl.ds(be_own, E), :],
                 out_ref.at[pl.ds(be_own, E), :], 7, y_nbr)
        b4a = mrc(out_ref.at[pl.ds(be_own, E), :],
                  out_ref.at[pl.ds(be_own, E), :], 10, x_nbr)
        b3.start()
        b4a.start()

        a3.wait()
        a4b = mrc(out_ref.at[pl.ds(ae_oth, E), :],
                  out_ref.at[pl.ds(ae_oth, E), :], 9, y_nbr)
        a4b.start()

        b3.wait()
        b4b = mrc(out_ref.at[pl.ds(be_oth, E), :],
                  out_ref.at[pl.ds(be_oth, E), :], 11, x_nbr)
        b4b.start()

        a4a.wait()
        b4a.wait()
        a4b.wait()
        b4b.wait()

    return pl.pallas_call(
        body,
        out_shape=jax.ShapeDtypeStruct((M, N), jnp.float32),
        in_specs=[pl.BlockSpec(memory_space=pltpu.VMEM)],
        out_specs=pl.BlockSpec(memory_space=pltpu.VMEM),
        scratch_shapes=[
            pltpu.VMEM((2, E, N), jnp.float32),
            pltpu.VMEM((2, E, N), jnp.float32),
            pltpu.VMEM((E, N), jnp.float32),
            pltpu.VMEM((E, N), jnp.float32),
            pltpu.SemaphoreType.DMA((12,)),
            pltpu.SemaphoreType.DMA((12,)),
        ],
        compiler_params=pltpu.CompilerParams(collective_id=0),
    )(x2d)


# device time: 17828 ns/iter; 1.7394x vs baseline; 1.0688x over previous
import jax
import jax.numpy as jnp
from jax import lax
from jax.experimental import pallas as pl
from jax.experimental.pallas import tpu as pltpu

M, N = 512, 512
H, Q, E = 256, 128, 64
K = 2
W = N // K


def kernel(x):
    x2d = x.reshape(M, N)

    def body(x_ref, out_ref, rA1, rB1, rA2, rB2, S, R):
        my_x = lax.axis_index("x")
        my_y = lax.axis_index("y")
        y_nbr = (my_x, 1 - my_y)
        x_nbr = (1 - my_x, my_y)

        barrier_sem = pltpu.get_barrier_semaphore()
        for nbr in (y_nbr, x_nbr):
            pl.semaphore_signal(
                barrier_sem, inc=1,
                device_id=nbr, device_id_type=pl.DeviceIdType.MESH,
            )
        pl.semaphore_wait(barrier_sem, 2)

        aq_own = my_y * Q
        aq_oth = (1 - my_y) * Q
        ae_own = aq_own + my_x * E
        ae_oth = aq_own + (1 - my_x) * E
        bq_own = H + my_x * Q
        bq_oth = H + (1 - my_x) * Q
        be_own = bq_own + my_y * E
        be_oth = bq_own + (1 - my_y) * E

        def mrc(src, dst, i, c, nbr):
            return pltpu.make_async_remote_copy(
                src_ref=src, dst_ref=dst,
                send_sem=S.at[i, c], recv_sem=R.at[i, c],
                device_id=nbr, device_id_type=pl.DeviceIdType.MESH,
            )

        def col(c):
            return pl.ds(c * W, W)

        a1f, a1s, b1f, b1s = [], [], [], []
        for c in range(K):
            a1f.append(mrc(x_ref.at[pl.ds(aq_oth + (1 - my_x) * E, E), col(c)],
                           rA1.at[0, :, col(c)], 0, c, y_nbr))
            b1f.append(mrc(x_ref.at[pl.ds(bq_oth + (1 - my_y) * E, E), col(c)],
                           rB1.at[0, :, col(c)], 1, c, x_nbr))
            a1s.append(mrc(x_ref.at[pl.ds(aq_oth + my_x * E, E), col(c)],
                           rA1.at[1, :, col(c)], 2, c, y_nbr))
            b1s.append(mrc(x_ref.at[pl.ds(bq_oth + my_y * E, E), col(c)],
                           rB1.at[1, :, col(c)], 3, c, x_nbr))
        for c in range(K):
            a1f[c].start()
            b1f[c].start()
        for c in range(K):
            a1s[c].start()
            b1s[c].start()

        a2, b2 = [], []
        for c in range(K):
            a1f[c].wait()
            out_ref[pl.ds(ae_oth, E), col(c)] = (
                x_ref[pl.ds(ae_oth, E), col(c)] + rA1[0, :, col(c)]
            )
            a2.append(mrc(out_ref.at[pl.ds(ae_oth, E), col(c)],
                          rA2.at[:, col(c)], 4, c, x_nbr))
            a2[c].start()
            b1f[c].wait()
            out_ref[pl.ds(be_oth, E), col(c)] = (
                x_ref[pl.ds(be_oth, E), col(c)] + rB1[0, :, col(c)]
            )
            b2.append(mrc(out_ref.at[pl.ds(be_oth, E), col(c)],
                          rB2.at[:, col(c)], 5, c, y_nbr))
            b2[c].start()

        for c in range(K):
            a1s[c].wait()
            out_ref[pl.ds(ae_own, E), col(c)] = (
                x_ref[pl.ds(ae_own, E), col(c)] + rA1[1, :, col(c)]
            )
            b1s[c].wait()
            out_ref[pl.ds(be_own, E), col(c)] = (
                x_ref[pl.ds(be_own, E), col(c)] + rB1[1, :, col(c)]
            )

        a3, b3, a4a, b4a = [], [], [], []
        for c in range(K):
            a2[c].wait()
            out_ref[pl.ds(ae_own, E), col(c)] = (
                out_ref[pl.ds(ae_own, E), col(c)] + rA2[:, col(c)]
            )
            a3.append(mrc(out_ref.at[pl.ds(ae_own, E), col(c)],
                          out_ref.at[pl.ds(ae_own, E), col(c)], 6, c, x_nbr))
            a4a.append(mrc(out_ref.at[pl.ds(ae_own, E), col(c)],
                           out_ref.at[pl.ds(ae_own, E), col(c)], 8, c, y_nbr))
            a3[c].start()
            a4a[c].start()
            b2[c].wait()
            out_ref[pl.ds(be_own, E), col(c)] = (
                out_ref[pl.ds(be_own, E), col(c)] + rB2[:, col(c)]
            )
            b3.append(mrc(out_ref.at[pl.ds(be_own, E), col(c)],
                          out_ref.at[pl.ds(be_own, E), col(c)], 7, c, y_nbr))
            b4a.append(mrc(out_ref.at[pl.ds(be_own, E), col(c)],
                           out_ref.at[pl.ds(be_own, E), col(c)], 10, c, x_nbr))
            b3[c].start()
            b4a[c].start()

        a4b, b4b = [], []
        for c in range(K):
            a3[c].wait()
            a4b.append(mrc(out_ref.at[pl.ds(ae_oth, E), col(c)],
                           out_ref.at[pl.ds(ae_oth, E), col(c)], 9, c, y_nbr))
            a4b[c].start()
            b3[c].wait()
            b4b.append(mrc(out_ref.at[pl.ds(be_oth, E), col(c)],
                           out_ref.at[pl.ds(be_oth, E), col(c)], 11, c, x_nbr))
            b4b[c].start()

        for c in range(K):
            a4a[c].wait()
            b4a[c].wait()
        for c in range(K):
            a4b[c].wait()
            b4b[c].wait()

    return pl.pallas_call(
        body,
        out_shape=jax.ShapeDtypeStruct((M, N), jnp.float32),
        in_specs=[pl.BlockSpec(memory_space=pltpu.VMEM)],
        out_specs=pl.BlockSpec(memory_space=pltpu.VMEM),
        scratch_shapes=[
            pltpu.VMEM((2, E, N), jnp.float32),
            pltpu.VMEM((2, E, N), jnp.float32),
            pltpu.VMEM((E, N), jnp.float32),
            pltpu.VMEM((E, N), jnp.float32),
            pltpu.SemaphoreType.DMA((12, K)),
            pltpu.SemaphoreType.DMA((12, K)),
        ],
        compiler_params=pltpu.CompilerParams(collective_id=0),
    )(x2d)


# device time: 17533 ns/iter; 1.7687x vs baseline; 1.0168x over previous
import jax
import jax.numpy as jnp
from jax import lax
from jax.experimental import pallas as pl
from jax.experimental.pallas import tpu as pltpu

M, N = 512, 512
H, Q, E = 256, 128, 64
K = 4
W = N // K


def kernel(x):
    x2d = x.reshape(M, N)

    def body(x_ref, out_ref, rA1, rB1, rA2, rB2, S, R):
        my_x = lax.axis_index("x")
        my_y = lax.axis_index("y")
        y_nbr = (my_x, 1 - my_y)
        x_nbr = (1 - my_x, my_y)

        barrier_sem = pltpu.get_barrier_semaphore()
        for nbr in (y_nbr, x_nbr):
            pl.semaphore_signal(
                barrier_sem, inc=1,
                device_id=nbr, device_id_type=pl.DeviceIdType.MESH,
            )
        pl.semaphore_wait(barrier_sem, 2)

        aq_own = my_y * Q
        aq_oth = (1 - my_y) * Q
        ae_own = aq_own + my_x * E
        ae_oth = aq_own + (1 - my_x) * E
        bq_own = H + my_x * Q
        bq_oth = H + (1 - my_x) * Q
        be_own = bq_own + my_y * E
        be_oth = bq_own + (1 - my_y) * E

        def mrc(src, dst, i, c, nbr):
            return pltpu.make_async_remote_copy(
                src_ref=src, dst_ref=dst,
                send_sem=S.at[i, c], recv_sem=R.at[i, c],
                device_id=nbr, device_id_type=pl.DeviceIdType.MESH,
            )

        def col(c):
            return pl.ds(c * W, W)

        a1f, a1s, b1f, b1s = [], [], [], []
        for c in range(K):
            a1f.append(mrc(x_ref.at[pl.ds(aq_oth + (1 - my_x) * E, E), col(c)],
                           rA1.at[0, :, col(c)], 0, c, y_nbr))
            b1f.append(mrc(x_ref.at[pl.ds(bq_oth + (1 - my_y) * E, E), col(c)],
                           rB1.at[0, :, col(c)], 1, c, x_nbr))
            a1s.append(mrc(x_ref.at[pl.ds(aq_oth + my_x * E, E), col(c)],
                           rA1.at[1, :, col(c)], 2, c, y_nbr))
            b1s.append(mrc(x_ref.at[pl.ds(bq_oth + my_y * E, E), col(c)],
                           rB1.at[1, :, col(c)], 3, c, x_nbr))
        for c in range(K):
            a1f[c].start()
            b1f[c].start()
        for c in range(K):
            a1s[c].start()
            b1s[c].start()

        a2, b2 = [], []
        for c in range(K):
            a1f[c].wait()
            out_ref[pl.ds(ae_oth, E), col(c)] = (
                x_ref[pl.ds(ae_oth, E), col(c)] + rA1[0, :, col(c)]
            )
            a2.append(mrc(out_ref.at[pl.ds(ae_oth, E), col(c)],
                          rA2.at[:, col(c)], 4, c, x_nbr))
            a2[c].start()
            b1f[c].wait()
            out_ref[pl.ds(be_oth, E), col(c)] = (
                x_ref[pl.ds(be_oth, E), col(c)] + rB1[0, :, col(c)]
            )
            b2.append(mrc(out_ref.at[pl.ds(be_oth, E), col(c)],
                          rB2.at[:, col(c)], 5, c, y_nbr))
            b2[c].start()

        for c in range(K):
            a1s[c].wait()
            out_ref[pl.ds(ae_own, E), col(c)] = (
                x_ref[pl.ds(ae_own, E), col(c)] + rA1[1, :, col(c)]
            )
            b1s[c].wait()
            out_ref[pl.ds(be_own, E), col(c)] = (
                x_ref[pl.ds(be_own, E), col(c)] + rB1[1, :, col(c)]
            )

        a3, b3, a4a, b4a = [], [], [], []
        for c in range(K):
            a2[c].wait()
            out_ref[pl.ds(ae_own, E), col(c)] = (
                out_ref[pl.ds(ae_own, E), col(c)] + rA2[:, col(c)]
            )
            a3.append(mrc(out_ref.at[pl.ds(ae_own, E), col(c)],
                          out_ref.at[pl.ds(ae_own, E), col(c)], 6, c, x_nbr))
            a4a.append(mrc(out_ref.at[pl.ds(ae_own, E), col(c)],
                           out_ref.at[pl.ds(ae_own, E), col(c)], 8, c, y_nbr))
            a3[c].start()
            a4a[c].start()
            b2[c].wait()
            out_ref[pl.ds(be_own, E), col(c)] = (
                out_ref[pl.ds(be_own, E), col(c)] + rB2[:, col(c)]
            )
            b3.append(mrc(out_ref.at[pl.ds(be_own, E), col(c)],
                          out_ref.at[pl.ds(be_own, E), col(c)], 7, c, y_nbr))
            b4a.append(mrc(out_ref.at[pl.ds(be_own, E), col(c)],
                           out_ref.at[pl.ds(be_own, E), col(c)], 10, c, x_nbr))
            b3[c].start()
            b4a[c].start()

        a4b, b4b = [], []
        for c in range(K):
            a3[c].wait()
            a4b.append(mrc(out_ref.at[pl.ds(ae_oth, E), col(c)],
                           out_ref.at[pl.ds(ae_oth, E), col(c)], 9, c, y_nbr))
            a4b[c].start()
            b3[c].wait()
            b4b.append(mrc(out_ref.at[pl.ds(be_oth, E), col(c)],
                           out_ref.at[pl.ds(be_oth, E), col(c)], 11, c, x_nbr))
            b4b[c].start()

        for c in range(K):
            a4a[c].wait()
            b4a[c].wait()
        for c in range(K):
            a4b[c].wait()
            b4b[c].wait()

    return pl.pallas_call(
        body,
        out_shape=jax.ShapeDtypeStruct((M, N), jnp.float32),
        in_specs=[pl.BlockSpec(memory_space=pltpu.VMEM)],
        out_specs=pl.BlockSpec(memory_space=pltpu.VMEM),
        scratch_shapes=[
            pltpu.VMEM((2, E, N), jnp.float32),
            pltpu.VMEM((2, E, N), jnp.float32),
            pltpu.VMEM((E, N), jnp.float32),
            pltpu.VMEM((E, N), jnp.float32),
            pltpu.SemaphoreType.DMA((12, K)),
            pltpu.SemaphoreType.DMA((12, K)),
        ],
        compiler_params=pltpu.CompilerParams(collective_id=0),
    )(x2d)


# device time: 16297 ns/iter; 1.9028x vs baseline; 1.0758x over previous
import jax
import jax.numpy as jnp
from jax import lax
from jax.experimental import pallas as pl
from jax.experimental.pallas import tpu as pltpu

M, N = 512, 512
H, Q = 256, 128
K = 2
W = N // K


def kernel(x):
    x2d = x.reshape(M, N)

    def body(x_ref, out_ref, rA1, rB1, rA2, rB2, S, R, xgate):
        my_x = lax.axis_index("x")
        my_y = lax.axis_index("y")
        y_nbr = (my_x, 1 - my_y)
        x_nbr = (1 - my_x, my_y)

        barrier_sem = pltpu.get_barrier_semaphore()
        pl.semaphore_signal(
            barrier_sem, inc=1,
            device_id=y_nbr, device_id_type=pl.DeviceIdType.MESH,
        )
        pl.semaphore_signal(
            xgate, inc=1,
            device_id=x_nbr, device_id_type=pl.DeviceIdType.MESH,
        )

        aq_own = my_y * Q
        aq_oth = (1 - my_y) * Q
        bq_own = H + my_x * Q
        bq_oth = H + (1 - my_x) * Q

        def mrc(src, dst, i, c, nbr):
            return pltpu.make_async_remote_copy(
                src_ref=src, dst_ref=dst,
                send_sem=S.at[i, c], recv_sem=R.at[i, c],
                device_id=nbr, device_id_type=pl.DeviceIdType.MESH,
            )

        def col(c):
            return pl.ds(c * W, W)

        a1 = [mrc(x_ref.at[pl.ds(aq_oth, Q), col(c)],
                  rA1.at[:, col(c)], 0, c, y_nbr) for c in range(K)]
        b1 = [mrc(x_ref.at[pl.ds(bq_oth, Q), col(c)],
                  rB1.at[:, col(c)], 1, c, x_nbr) for c in range(K)]

        pl.semaphore_wait(barrier_sem, 1)
        for c in range(K):
            a1[c].start()
        pl.semaphore_wait(xgate, 1)
        for c in range(K):
            b1[c].start()

        a2, b2 = [], []
        for c in range(K):
            a1[c].wait()
            out_ref[pl.ds(aq_own, Q), col(c)] = (
                x_ref[pl.ds(aq_own, Q), col(c)] + rA1[:, col(c)]
            )
            a2.append(mrc(out_ref.at[pl.ds(aq_own, Q), col(c)],
                          rA2.at[:, col(c)], 2, c, x_nbr))
            a2[c].start()
            b1[c].wait()
            out_ref[pl.ds(bq_own, Q), col(c)] = (
                x_ref[pl.ds(bq_own, Q), col(c)] + rB1[:, col(c)]
            )
            b2.append(mrc(out_ref.at[pl.ds(bq_own, Q), col(c)],
                          rB2.at[:, col(c)], 3, c, y_nbr))
            b2[c].start()

        a3, b3 = [], []
        for c in range(K):
            a2[c].wait()
            out_ref[pl.ds(aq_own, Q), col(c)] = (
                out_ref[pl.ds(aq_own, Q), col(c)] + rA2[:, col(c)]
            )
            a3.append(mrc(out_ref.at[pl.ds(aq_own, Q), col(c)],
                          out_ref.at[pl.ds(aq_own, Q), col(c)], 4, c, y_nbr))
            a3[c].start()
            b2[c].wait()
            out_ref[pl.ds(bq_own, Q), col(c)] = (
                out_ref[pl.ds(bq_own, Q), col(c)] + rB2[:, col(c)]
            )
            b3.append(mrc(out_ref.at[pl.ds(bq_own, Q), col(c)],
                          out_ref.at[pl.ds(bq_own, Q), col(c)], 5, c, x_nbr))
            b3[c].start()

        for c in range(K):
            a3[c].wait()
            b3[c].wait()

    return pl.pallas_call(
        body,
        out_shape=jax.ShapeDtypeStruct((M, N), jnp.float32),
        in_specs=[pl.BlockSpec(memory_space=pltpu.VMEM)],
        out_specs=pl.BlockSpec(memory_space=pltpu.VMEM),
        scratch_shapes=[
            pltpu.VMEM((Q, N), jnp.float32),
            pltpu.VMEM((Q, N), jnp.float32),
            pltpu.VMEM((Q, N), jnp.float32),
            pltpu.VMEM((Q, N), jnp.float32),
            pltpu.SemaphoreType.DMA((6, K)),
            pltpu.SemaphoreType.DMA((6, K)),
            pltpu.SemaphoreType.REGULAR,
        ],
        compiler_params=pltpu.CompilerParams(collective_id=0),
    )(x2d)


# device time: 15920 ns/iter; 1.9479x vs baseline; 1.0237x over previous
import jax
import jax.numpy as jnp
from jax import lax
from jax.experimental import pallas as pl
from jax.experimental.pallas import tpu as pltpu

M, N = 512, 512
H, Q = 256, 128
CS = [32, 32, 32, 32]
CO = [0, 32, 64, 96]
R = len(CS)


def kernel(x):
    def body(x4_ref, out_ref, rA1, rB1, rA2, rB2, S, Rs, xgate):
        x_ref = x4_ref.at[0, 0]
        my_x = lax.axis_index("x")
        my_y = lax.axis_index("y")
        y_nbr = (my_x, 1 - my_y)
        x_nbr = (1 - my_x, my_y)

        barrier_sem = pltpu.get_barrier_semaphore()
        pl.semaphore_signal(
            barrier_sem, inc=1,
            device_id=y_nbr, device_id_type=pl.DeviceIdType.MESH,
        )
        pl.semaphore_signal(
            xgate, inc=1,
            device_id=x_nbr, device_id_type=pl.DeviceIdType.MESH,
        )

        aq_own = my_y * Q
        aq_oth = (1 - my_y) * Q
        bq_own = H + my_x * Q
        bq_oth = H + (1 - my_x) * Q

        def mrc(src, dst, i, c, nbr):
            return pltpu.make_async_remote_copy(
                src_ref=src, dst_ref=dst,
                send_sem=S.at[i, c], recv_sem=Rs.at[i, c],
                device_id=nbr, device_id_type=pl.DeviceIdType.MESH,
            )

        def rows(base, c):
            return pl.ds(base + CO[c], CS[c])

        a1 = [mrc(x_ref.at[rows(aq_oth, c), :],
                  rA1.at[rows(0, c), :], 0, c, y_nbr) for c in range(R)]
        b1 = [mrc(x_ref.at[rows(bq_oth, c), :],
                  rB1.at[rows(0, c), :], 1, c, x_nbr) for c in range(R)]

        pl.semaphore_wait(barrier_sem, 1)
        for c in range(R):
            a1[c].start()
        pl.semaphore_wait(xgate, 1)
        for c in range(R):
            b1[c].start()

        a2, b2 = [], []
        for c in range(R):
            a1[c].wait()
            out_ref[rows(aq_own, c), :] = (
                x_ref[rows(aq_own, c), :] + rA1[rows(0, c), :]
            )
            a2.append(mrc(out_ref.at[rows(aq_own, c), :],
                          rA2.at[rows(0, c), :], 2, c, x_nbr))
            a2[c].start()
            b1[c].wait()
            out_ref[rows(bq_own, c), :] = (
                x_ref[rows(bq_own, c), :] + rB1[rows(0, c), :]
            )
            b2.append(mrc(out_ref.at[rows(bq_own, c), :],
                          rB2.at[rows(0, c), :], 3, c, y_nbr))
            b2[c].start()

        a3, b3 = [], []
        for c in range(R):
            a2[c].wait()
            out_ref[rows(aq_own, c), :] = (
                out_ref[rows(aq_own, c), :] + rA2[rows(0, c), :]
            )
            a3.append(mrc(out_ref.at[rows(aq_own, c), :],
                          out_ref.at[rows(aq_own, c), :], 4, c, y_nbr))
            a3[c].start()
            b2[c].wait()
            out_ref[rows(bq_own, c), :] = (
                out_ref[rows(bq_own, c), :] + rB2[rows(0, c), :]
            )
            b3.append(mrc(out_ref.at[rows(bq_own, c), :],
                          out_ref.at[rows(bq_own, c), :], 5, c, x_nbr))
            b3[c].start()

        for c in range(R):
            a3[c].wait()
            b3[c].wait()

    return pl.pallas_call(
        body,
        out_shape=jax.ShapeDtypeStruct((M, N), jnp.float32),
        in_specs=[pl.BlockSpec(memory_space=pltpu.VMEM)],
        out_specs=pl.BlockSpec(memory_space=pltpu.VMEM),
        scratch_shapes=[
            pltpu.VMEM((Q, N), jnp.float32),
            pltpu.VMEM((Q, N), jnp.float32),
            pltpu.VMEM((Q, N), jnp.float32),
            pltpu.VMEM((Q, N), jnp.float32),
            pltpu.SemaphoreType.DMA((6, R)),
            pltpu.SemaphoreType.DMA((6, R)),
            pltpu.SemaphoreType.REGULAR,
        ],
        compiler_params=pltpu.CompilerParams(collective_id=0),
    )(x)
